# baseline (speedup 1.0000x reference)
import functools

import jax
import jax.numpy as jnp
from jax import lax
from jax.experimental import pallas as pl
from jax.experimental.pallas import tpu as pltpu
from jax.experimental.pallas import tpu_sc as plsc

N_FIELDS = 26
VOCAB1 = 100001
EMB = 64
BATCH = 4096
NW = 32
CHUNK = BATCH // NW
LANES = 16

_mesh = plsc.VectorSubcoreMesh(core_axis_name="c", subcore_axis_name="s")


@functools.partial(
    pl.kernel,
    mesh=_mesh,
    compiler_params=pltpu.CompilerParams(use_tc_tiling_on_sc=False),
    out_type=jax.ShapeDtypeStruct((N_FIELDS * BATCH, EMB), jnp.float32),
    scratch_types=[
        pltpu.VMEM((CHUNK,), jnp.int32),
        pltpu.VMEM((CHUNK, EMB), jnp.float32),
        pltpu.SemaphoreType.DMA,
    ],
)
def _gather_kernel(xs_hbm, tab_hbm, out_hbm, idx_v, rows_v, sem):
    wid = lax.axis_index("s") * 2 + lax.axis_index("c")
    col0 = wid * CHUNK

    def body(f, carry):
        base = f * BATCH + col0
        pltpu.sync_copy(xs_hbm.at[pl.ds(base, CHUNK)], idx_v)
        off = f * VOCAB1
        for i in range(CHUNK // LANES):
            sl = pl.ds(i * LANES, LANES)
            idx_v[sl] = idx_v[sl] + off
        pltpu.async_copy(tab_hbm.at[idx_v], rows_v, sem).wait()
        pltpu.sync_copy(rows_v, out_hbm.at[pl.ds(base, CHUNK)])
        return carry

    lax.fori_loop(0, N_FIELDS, body, 0)


def kernel(xs, W):
    out = _gather_kernel(xs.reshape(N_FIELDS * BATCH), W.reshape(N_FIELDS * VOCAB1, EMB))
    return out.reshape(N_FIELDS, BATCH, EMB)
